# Initial kernel scaffold; baseline (speedup 1.0000x reference)
import jax
import jax.numpy as jnp
from jax.experimental import pallas as pl


def kernel(x, edge_index, edge_attr, W_ae, b_ae, g_ae, bb_ae, Wl, bl, root, Wbe, b_be, g_be, bb_be, Wih, Whh, bih, bhh, g_bn, bb_bn):
    raise NotImplementedError("write your pallas kernel here")


# speedup vs baseline: 3.6423x; 3.6423x over previous
import functools
import jax
import jax.numpy as jnp
from jax import lax
from jax.experimental import pallas as pl
from jax.experimental.pallas import tpu as pltpu
from jax.experimental.pallas import tpu_sc as plsc

N = 10000
E = 320000
D = 128
DE = 16
L = 2

NC = 2
NS = 16
NW = NC * NS
EPT = E // NW
C = 80
NCHUNK = EPT // C
R8 = 624
TAIL = N - R8 * NS

_MESH = plsc.VectorSubcoreMesh(core_axis_name="c", subcore_axis_name="s")
_SC_PARAMS = pltpu.CompilerParams(needs_layout_passes=False)
_TC_PARAMS = pltpu.CompilerParams(vmem_limit_bytes=100 * 1024 * 1024)


def _stripe_copy(src_at, dst_at, s):
    off = pl.multiple_of(s * R8, 8)
    pltpu.sync_copy(src_at(off, R8), dst_at(off, R8))

    @pl.when(s == NS - 1)
    def _():
        pltpu.sync_copy(src_at(R8 * NS, TAIL), dst_at(R8 * NS, TAIL))


@functools.partial(
    pl.kernel,
    mesh=_MESH,
    out_type=jax.ShapeDtypeStruct((NC, N, D), jnp.float32),
    scratch_types=[
        pltpu.VMEM((C,), jnp.int32),
        pltpu.VMEM((C, D), jnp.float32),
        pltpu.VMEM_SHARED((N, D), jnp.float32),
    ],
    compiler_params=_SC_PARAMS,
)
def _deg_kernel(row_hbm, zeros_hbm, out_hbm, rowb, onesb, deg_sh):
    c = lax.axis_index("c")
    s = lax.axis_index("s")
    wid = c * NS + s
    _stripe_copy(lambda o, n: zeros_hbm.at[pl.ds(o, n)],
                 lambda o, n: deg_sh.at[pl.ds(o, n)], s)
    one = jnp.full((16,), 1.0, jnp.float32)

    def fill(i, carry):
        for d in range(D // 16):
            onesb[i, pl.ds(d * 16, 16)] = one
        return carry

    lax.fori_loop(0, C, fill, 0)
    plsc.subcore_barrier()

    def chunk(j, carry):
        base = pl.multiple_of(wid * EPT + j * C, 8)
        pltpu.sync_copy(row_hbm.at[pl.ds(base, C)], rowb)
        pltpu.sync_copy(onesb, deg_sh.at[rowb], add=True)
        return carry

    lax.fori_loop(0, NCHUNK, chunk, 0)
    plsc.subcore_barrier()
    _stripe_copy(lambda o, n: deg_sh.at[pl.ds(o, n)],
                 lambda o, n: out_hbm.at[c, pl.ds(o, n)], s)


@functools.partial(
    pl.kernel,
    mesh=_MESH,
    out_type=jax.ShapeDtypeStruct((NC, N, D), jnp.float32),
    scratch_types=[
        pltpu.VMEM((C,), jnp.int32),
        pltpu.VMEM((C,), jnp.int32),
        pltpu.VMEM((C,), jnp.float32),
        pltpu.VMEM((N,), jnp.float32),
        pltpu.VMEM((C, D), jnp.float32),
        pltpu.VMEM((C, D), jnp.float32),
        pltpu.SemaphoreType.DMA,
        pltpu.VMEM_SHARED((N, D), jnp.float32),
    ],
    compiler_params=_SC_PARAMS,
)
def _edge_kernel(xl_hbm, ee_hbm, row_hbm, col_hbm, dinv_hbm, zeros_hbm,
                 out_hbm, rowb, colb, normb, dinvv, xg, el, sem, aggr_sh):
    c = lax.axis_index("c")
    s = lax.axis_index("s")
    wid = c * NS + s
    _stripe_copy(lambda o, n: zeros_hbm.at[pl.ds(o, n)],
                 lambda o, n: aggr_sh.at[pl.ds(o, n)], s)
    pltpu.sync_copy(dinv_hbm, dinvv)
    plsc.subcore_barrier()

    def chunk(j, carry):
        base = pl.multiple_of(wid * EPT + j * C, 8)
        pltpu.sync_copy(row_hbm.at[pl.ds(base, C)], rowb)
        pltpu.sync_copy(col_hbm.at[pl.ds(base, C)], colb)
        pltpu.async_copy(xl_hbm.at[rowb], xg, sem).wait()
        pltpu.sync_copy(ee_hbm.at[pl.ds(base, C)], el)

        def nloop(k, cy):
            iv_r = rowb[pl.ds(k * 16, 16)]
            iv_c = colb[pl.ds(k * 16, 16)]
            dr = plsc.load_gather(dinvv, [iv_r])
            dc = plsc.load_gather(dinvv, [iv_c])
            normb[pl.ds(k * 16, 16)] = dr * dc
            return cy

        lax.fori_loop(0, C // 16, nloop, 0)

        def eloop(e, cy):
            nv = plsc.load_gather(normb, [jnp.full((16,), e, jnp.int32)])
            for d in range(D // 16):
                v = xg[e, pl.ds(d * 16, 16)] + el[e, pl.ds(d * 16, 16)]
                xg[e, pl.ds(d * 16, 16)] = jnp.maximum(v, 0.0) * nv
            return cy

        lax.fori_loop(0, C, eloop, 0)
        pltpu.sync_copy(xg, aggr_sh.at[colb], add=True)
        return carry

    lax.fori_loop(0, NCHUNK, chunk, 0)
    plsc.subcore_barrier()
    _stripe_copy(lambda o, n: aggr_sh.at[pl.ds(o, n)],
                 lambda o, n: out_hbm.at[c, pl.ds(o, n)], s)


def _prep_body(x_ref, b_ref, wae_ref, bae_ref, gae_ref, bbae_ref,
               wl0_ref, bl0_ref, wbe_ref, bbe_ref, gbe_ref, bbbe_ref,
               parts_ref, xl0_ref, deg_ref, dinv_ref, wf_ref, cf_ref):
    f32 = jnp.float32
    x = x_ref[...]
    y = jnp.dot(x, wae_ref[...], preferred_element_type=f32) + bae_ref[...]
    m = jnp.mean(y, axis=0, keepdims=True)
    v = jnp.mean((y - m) ** 2, axis=0, keepdims=True)
    h0 = (y - m) / jnp.sqrt(v + 1e-6) * gae_ref[...] + bbae_ref[...]
    xl0_ref[...] = jnp.dot(h0, wl0_ref[...], preferred_element_type=f32) + bl0_ref[...]

    parts = parts_ref[...]
    deg = parts[0, :, 0:1] + parts[1, :, 0:1] + 1.0
    deg_ref[...] = deg
    dinv_ref[...] = lax.rsqrt(deg)

    bmat = b_ref[...]
    bb = lax.dot_general(bmat, bmat, (((0,), (0,)), ((), ())),
                         preferred_element_type=f32)
    ones_row = jnp.full((1, bmat.shape[0]), 1.0, f32)
    csum = jnp.dot(ones_row, bmat, preferred_element_type=f32)
    m2 = jnp.zeros((16, 16), f32)
    asum = jnp.zeros((1, 16), f32)
    for i in range(8):
        m2 = m2 + bb[i * 16:(i + 1) * 16, i * 16:(i + 1) * 16]
        asum = asum + csum[:, i * 16:(i + 1) * 16]
    abar = asum * (1.0 / E)
    cov = m2 * (1.0 / E) - lax.dot_general(
        abar, abar, (((0,), (0,)), ((), ())), preferred_element_type=f32)

    for l in range(L):
        w = wbe_ref[l]
        mu = jnp.dot(abar, w, preferred_element_type=f32) + bbe_ref[l:l + 1, :]
        cw = jnp.dot(cov, w, preferred_element_type=f32)
        var = jnp.sum(w * cw, axis=0, keepdims=True)
        sc = gbe_ref[l:l + 1, :] / jnp.sqrt(var + 1e-6)
        wf_ref[l] = w * sc
        cf_ref[l:l + 1, :] = (bbe_ref[l:l + 1, :] - mu) * sc + bbbe_ref[l:l + 1, :]


EB = 8000


def _ee_body(a_ref, wf_ref, cf_ref, ee0_ref, ee1_ref):
    f32 = jnp.float32
    a = a_ref[...]
    ee0_ref[...] = jnp.dot(a, wf_ref[0], preferred_element_type=f32) + cf_ref[0:1, :]
    ee1_ref[...] = jnp.dot(a, wf_ref[1], preferred_element_type=f32) + cf_ref[1:2, :]


def _make_upd_body(last):
    def body(parts_ref, xl_ref, wih_ref, whh_ref, bih_ref, bhh_ref,
             root_ref, deg_ref, gbn_ref, bbbn_ref, *rest):
        f32 = jnp.float32
        if last:
            (out_ref,) = rest
        else:
            wl1_ref, bl1_ref, out_ref = rest
        parts = parts_ref[...]
        aggr = parts[0] + parts[1]
        xl = xl_ref[...]
        gi = jnp.dot(aggr, wih_ref[...], preferred_element_type=f32) + bih_ref[...]
        gh = jnp.dot(xl, whh_ref[...], preferred_element_type=f32) + bhh_ref[...]
        r = jax.nn.sigmoid(gi[:, 0:D] + gh[:, 0:D])
        z = jax.nn.sigmoid(gi[:, D:2 * D] + gh[:, D:2 * D])
        n = jnp.tanh(gi[:, 2 * D:3 * D] + r * gh[:, 2 * D:3 * D])
        upd = (1.0 - z) * n + z * xl
        conv = upd + jnp.maximum(xl + root_ref[...], 0.0) / deg_ref[...]
        m = jnp.mean(conv, axis=0, keepdims=True)
        v = jnp.mean((conv - m) ** 2, axis=0, keepdims=True)
        hb = (conv - m) / jnp.sqrt(v + 1e-5) * gbn_ref[...] + bbbn_ref[...]
        if last:
            out_ref[...] = hb
        else:
            h = jnp.maximum(hb, 0.0)
            out_ref[...] = jnp.dot(h, wl1_ref[...], preferred_element_type=f32) + bl1_ref[...]
    return body


def kernel(x, edge_index, edge_attr, W_ae, b_ae, g_ae, bb_ae, Wl, bl, root,
           Wbe, b_be, g_be, bb_be, Wih, Whh, bih, bhh, g_bn, bb_bn):
    f32 = jnp.float32
    zerosD = jnp.zeros((N, D), f32)
    bpacked = edge_attr.reshape(E * DE // 128, 128)
    row = edge_index[0]
    col = edge_index[1]

    deg_parts = _deg_kernel(row, zerosD)

    xl0, deg, dinv2, Wf, cf = pl.pallas_call(
        _prep_body,
        out_shape=(
            jax.ShapeDtypeStruct((N, D), f32),
            jax.ShapeDtypeStruct((N, 1), f32),
            jax.ShapeDtypeStruct((N, 1), f32),
            jax.ShapeDtypeStruct((L, DE, D), f32),
            jax.ShapeDtypeStruct((L, D), f32),
        ),
        compiler_params=_TC_PARAMS,
    )(x, bpacked, W_ae, b_ae.reshape(1, D), g_ae.reshape(1, D),
      bb_ae.reshape(1, D), Wl[0], bl[0].reshape(1, D), Wbe, b_be, g_be,
      bb_be, deg_parts)
    dinv = dinv2.reshape(N)

    ee0, ee1 = pl.pallas_call(
        _ee_body,
        grid=(E // EB,),
        in_specs=[
            pl.BlockSpec((EB, DE), lambda i: (i, 0)),
            pl.BlockSpec((L, DE, D), lambda i: (0, 0, 0)),
            pl.BlockSpec((L, D), lambda i: (0, 0)),
        ],
        out_specs=(
            pl.BlockSpec((EB, D), lambda i: (i, 0)),
            pl.BlockSpec((EB, D), lambda i: (i, 0)),
        ),
        out_shape=(
            jax.ShapeDtypeStruct((E, D), f32),
            jax.ShapeDtypeStruct((E, D), f32),
        ),
    )(edge_attr, Wf, cf)

    ees = (ee0, ee1)
    xl = xl0
    for l in range(L):
        aggr_parts = _edge_kernel(xl, ees[l], row, col, dinv, zerosD)
        last = l == L - 1
        ops = [aggr_parts, xl, Wih[l], Whh[l], bih[l].reshape(1, 3 * D),
               bhh[l].reshape(1, 3 * D), root[l].reshape(1, D), deg,
               g_bn[l].reshape(1, D), bb_bn[l].reshape(1, D)]
        if not last:
            ops += [Wl[1], bl[1].reshape(1, D)]
        xl = pl.pallas_call(
            _make_upd_body(last),
            out_shape=jax.ShapeDtypeStruct((N, D), f32),
            compiler_params=_TC_PARAMS,
        )(*ops)
    return xl


# speedup vs baseline: 4.6779x; 1.2843x over previous
import functools
import jax
import jax.numpy as jnp
from jax import lax
from jax.experimental import pallas as pl
from jax.experimental.pallas import tpu as pltpu
from jax.experimental.pallas import tpu_sc as plsc

N = 10000
E = 320000
D = 128
DE = 16
L = 2

NC = 2
NS = 16
NW = NC * NS
EPT = E // NW
C = 80
NCHUNK = EPT // C
R8 = 624
TAIL = N - R8 * NS

_MESH = plsc.VectorSubcoreMesh(core_axis_name="c", subcore_axis_name="s")
_SC_PARAMS = pltpu.CompilerParams(needs_layout_passes=False)
_TC_PARAMS = pltpu.CompilerParams(vmem_limit_bytes=100 * 1024 * 1024)


def _stripe_copy(src_at, dst_at, s):
    off = pl.multiple_of(s * R8, 8)
    pltpu.sync_copy(src_at(off, R8), dst_at(off, R8))

    @pl.when(s == NS - 1)
    def _():
        pltpu.sync_copy(src_at(R8 * NS, TAIL), dst_at(R8 * NS, TAIL))


@functools.partial(
    pl.kernel,
    mesh=_MESH,
    out_type=jax.ShapeDtypeStruct((NC, N, D), jnp.float32),
    scratch_types=[
        pltpu.VMEM((C,), jnp.int32),
        pltpu.VMEM((C, D), jnp.float32),
        pltpu.VMEM_SHARED((N, D), jnp.float32),
    ],
    compiler_params=_SC_PARAMS,
)
def _deg_kernel(row_hbm, zeros_hbm, out_hbm, rowb, onesb, deg_sh):
    c = lax.axis_index("c")
    s = lax.axis_index("s")
    wid = c * NS + s
    _stripe_copy(lambda o, n: zeros_hbm.at[pl.ds(o, n)],
                 lambda o, n: deg_sh.at[pl.ds(o, n)], s)
    one = jnp.full((16,), 1.0, jnp.float32)

    def fill(i, carry):
        for d in range(D // 16):
            onesb[i, pl.ds(d * 16, 16)] = one
        return carry

    lax.fori_loop(0, C, fill, 0)
    plsc.subcore_barrier()

    def chunk(j, carry):
        base = pl.multiple_of(wid * EPT + j * C, 8)
        pltpu.sync_copy(row_hbm.at[pl.ds(base, C)], rowb)
        pltpu.sync_copy(onesb, deg_sh.at[rowb], add=True)
        return carry

    lax.fori_loop(0, NCHUNK, chunk, 0)
    plsc.subcore_barrier()
    _stripe_copy(lambda o, n: deg_sh.at[pl.ds(o, n)],
                 lambda o, n: out_hbm.at[c, pl.ds(o, n)], s)


NB = 2000


@functools.partial(
    pl.kernel,
    mesh=_MESH,
    out_type=jax.ShapeDtypeStruct((E,), jnp.float32),
    scratch_types=[
        pltpu.VMEM((NB,), jnp.int32),
        pltpu.VMEM((NB,), jnp.int32),
        pltpu.VMEM((NB,), jnp.float32),
        pltpu.VMEM((N,), jnp.float32),
    ],
    compiler_params=_SC_PARAMS,
)
def _norm_kernel(row_hbm, col_hbm, dinv_hbm, out_hbm, rowb, colb, normb, dinvv):
    c = lax.axis_index("c")
    s = lax.axis_index("s")
    wid = c * NS + s
    pltpu.sync_copy(dinv_hbm, dinvv)

    def blk(j, carry):
        base = pl.multiple_of(wid * EPT + j * NB, 8)
        pltpu.sync_copy(row_hbm.at[pl.ds(base, NB)], rowb)
        pltpu.sync_copy(col_hbm.at[pl.ds(base, NB)], colb)

        def nloop(k, cy):
            dr = plsc.load_gather(dinvv, [rowb[pl.ds(k * 16, 16)]])
            dc = plsc.load_gather(dinvv, [colb[pl.ds(k * 16, 16)]])
            normb[pl.ds(k * 16, 16)] = dr * dc
            return cy

        lax.fori_loop(0, NB // 16, nloop, 0)
        pltpu.sync_copy(normb, out_hbm.at[pl.ds(base, NB)])
        return carry

    lax.fori_loop(0, EPT // NB, blk, 0)


@functools.partial(
    pl.kernel,
    mesh=_MESH,
    out_type=jax.ShapeDtypeStruct((NC, N, D), jnp.float32),
    scratch_types=[
        [pltpu.VMEM((C,), jnp.int32)] * 2,
        [pltpu.VMEM((C,), jnp.int32)] * 2,
        [pltpu.VMEM((C,), jnp.float32)] * 2,
        [pltpu.VMEM((C, D), jnp.float32)] * 2,
        [pltpu.VMEM((C, D), jnp.float32)] * 2,
        [pltpu.SemaphoreType.DMA] * 2,
        [pltpu.SemaphoreType.DMA] * 2,
        [pltpu.SemaphoreType.DMA] * 2,
        [pltpu.SemaphoreType.DMA] * 2,
        pltpu.VMEM_SHARED((N, D), jnp.float32),
    ],
    compiler_params=_SC_PARAMS,
)
def _edge_kernel(xl_hbm, ee_hbm, row_hbm, col_hbm, norm_hbm, zeros_hbm,
                 out_hbm, rowb, colb, normb, xg, el, gsem, esem, nsem,
                 ssem, aggr_sh):
    c = lax.axis_index("c")
    s = lax.axis_index("s")
    wid = c * NS + s
    _stripe_copy(lambda o, n: zeros_hbm.at[pl.ds(o, n)],
                 lambda o, n: aggr_sh.at[pl.ds(o, n)], s)
    plsc.subcore_barrier()

    def drain(b):
        pltpu.make_async_copy(xg[b], aggr_sh.at[colb[b]], ssem[b]).wait()

    def stage(j, b):
        base = pl.multiple_of(wid * EPT + j * C, 8)
        pltpu.sync_copy(row_hbm.at[pl.ds(base, C)], rowb[b])
        pltpu.sync_copy(col_hbm.at[pl.ds(base, C)], colb[b])
        pltpu.async_copy(xl_hbm.at[rowb[b]], xg[b], gsem[b])
        pltpu.async_copy(ee_hbm.at[pl.ds(base, C)], el[b], esem[b])
        pltpu.async_copy(norm_hbm.at[pl.ds(base, C)], normb[b], nsem[b])

    def front(j, b):
        drain(b)
        stage(j, b)

    def back(b):
        pltpu.make_async_copy(xl_hbm.at[rowb[b]], xg[b], gsem[b]).wait()
        pltpu.make_async_copy(ee_hbm.at[pl.ds(0, C)], el[b], esem[b]).wait()
        pltpu.make_async_copy(norm_hbm.at[pl.ds(0, C)], normb[b], nsem[b]).wait()

        def eloop(e, cy):
            nv = plsc.load_gather(normb[b], [jnp.full((16,), e, jnp.int32)])
            for d in range(D // 16):
                v = xg[b][e, pl.ds(d * 16, 16)] + el[b][e, pl.ds(d * 16, 16)]
                xg[b][e, pl.ds(d * 16, 16)] = jnp.maximum(v, 0.0) * nv
            return cy

        lax.fori_loop(0, C, eloop, 0)
        pltpu.async_copy(xg[b], aggr_sh.at[colb[b]], ssem[b], add=True)

    stage(0, 0)
    stage(1, 1)
    back(0)
    front(2, 0)
    back(1)

    def pair(i, carry):
        front(2 * i + 3, 1)
        back(0)
        front(2 * i + 4, 0)
        back(1)
        return carry

    lax.fori_loop(0, (NCHUNK - 3) // 2, pair, 0)
    back(0)
    drain(1)
    drain(0)
    plsc.subcore_barrier()
    _stripe_copy(lambda o, n: aggr_sh.at[pl.ds(o, n)],
                 lambda o, n: out_hbm.at[c, pl.ds(o, n)], s)


def _prep_body(x_ref, b_ref, wae_ref, bae_ref, gae_ref, bbae_ref,
               wl0_ref, bl0_ref, wbe_ref, bbe_ref, gbe_ref, bbbe_ref,
               parts_ref, xl0_ref, deg_ref, dinv_ref, wf_ref, cf_ref):
    f32 = jnp.float32
    x = x_ref[...]
    y = jnp.dot(x, wae_ref[...], preferred_element_type=f32) + bae_ref[...]
    m = jnp.mean(y, axis=0, keepdims=True)
    v = jnp.mean((y - m) ** 2, axis=0, keepdims=True)
    h0 = (y - m) / jnp.sqrt(v + 1e-6) * gae_ref[...] + bbae_ref[...]
    xl0_ref[...] = jnp.dot(h0, wl0_ref[...], preferred_element_type=f32) + bl0_ref[...]

    parts = parts_ref[...]
    deg = parts[0, :, 0:1] + parts[1, :, 0:1] + 1.0
    deg_ref[...] = deg
    dinv_ref[...] = lax.rsqrt(deg)

    bmat = b_ref[...]
    bb = lax.dot_general(bmat, bmat, (((0,), (0,)), ((), ())),
                         preferred_element_type=f32)
    ones_row = jnp.full((1, bmat.shape[0]), 1.0, f32)
    csum = jnp.dot(ones_row, bmat, preferred_element_type=f32)
    m2 = jnp.zeros((16, 16), f32)
    asum = jnp.zeros((1, 16), f32)
    for i in range(8):
        m2 = m2 + bb[i * 16:(i + 1) * 16, i * 16:(i + 1) * 16]
        asum = asum + csum[:, i * 16:(i + 1) * 16]
    abar = asum * (1.0 / E)
    cov = m2 * (1.0 / E) - lax.dot_general(
        abar, abar, (((0,), (0,)), ((), ())), preferred_element_type=f32)

    for l in range(L):
        w = wbe_ref[l]
        mu = jnp.dot(abar, w, preferred_element_type=f32) + bbe_ref[l:l + 1, :]
        cw = jnp.dot(cov, w, preferred_element_type=f32)
        var = jnp.sum(w * cw, axis=0, keepdims=True)
        sc = gbe_ref[l:l + 1, :] / jnp.sqrt(var + 1e-6)
        wf_ref[l] = w * sc
        cf_ref[l:l + 1, :] = (bbe_ref[l:l + 1, :] - mu) * sc + bbbe_ref[l:l + 1, :]


EB = 8000


def _ee_body(a_ref, wf_ref, cf_ref, ee0_ref, ee1_ref):
    f32 = jnp.float32
    a = a_ref[...]
    ee0_ref[...] = jnp.dot(a, wf_ref[0], preferred_element_type=f32) + cf_ref[0:1, :]
    ee1_ref[...] = jnp.dot(a, wf_ref[1], preferred_element_type=f32) + cf_ref[1:2, :]


def _make_upd_body(last):
    def body(parts_ref, xl_ref, wih_ref, whh_ref, bih_ref, bhh_ref,
             root_ref, deg_ref, gbn_ref, bbbn_ref, *rest):
        f32 = jnp.float32
        if last:
            (out_ref,) = rest
        else:
            wl1_ref, bl1_ref, out_ref = rest
        parts = parts_ref[...]
        aggr = parts[0] + parts[1]
        xl = xl_ref[...]
        gi = jnp.dot(aggr, wih_ref[...], preferred_element_type=f32) + bih_ref[...]
        gh = jnp.dot(xl, whh_ref[...], preferred_element_type=f32) + bhh_ref[...]
        r = jax.nn.sigmoid(gi[:, 0:D] + gh[:, 0:D])
        z = jax.nn.sigmoid(gi[:, D:2 * D] + gh[:, D:2 * D])
        n = jnp.tanh(gi[:, 2 * D:3 * D] + r * gh[:, 2 * D:3 * D])
        upd = (1.0 - z) * n + z * xl
        conv = upd + jnp.maximum(xl + root_ref[...], 0.0) / deg_ref[...]
        m = jnp.mean(conv, axis=0, keepdims=True)
        v = jnp.mean((conv - m) ** 2, axis=0, keepdims=True)
        hb = (conv - m) / jnp.sqrt(v + 1e-5) * gbn_ref[...] + bbbn_ref[...]
        if last:
            out_ref[...] = hb
        else:
            h = jnp.maximum(hb, 0.0)
            out_ref[...] = jnp.dot(h, wl1_ref[...], preferred_element_type=f32) + bl1_ref[...]
    return body


def kernel(x, edge_index, edge_attr, W_ae, b_ae, g_ae, bb_ae, Wl, bl, root,
           Wbe, b_be, g_be, bb_be, Wih, Whh, bih, bhh, g_bn, bb_bn):
    f32 = jnp.float32
    zerosD = jnp.zeros((N, D), f32)
    bpacked = edge_attr.reshape(E * DE // 128, 128)
    row = edge_index[0]
    col = edge_index[1]

    deg_parts = _deg_kernel(row, zerosD)

    xl0, deg, dinv2, Wf, cf = pl.pallas_call(
        _prep_body,
        out_shape=(
            jax.ShapeDtypeStruct((N, D), f32),
            jax.ShapeDtypeStruct((N, 1), f32),
            jax.ShapeDtypeStruct((N, 1), f32),
            jax.ShapeDtypeStruct((L, DE, D), f32),
            jax.ShapeDtypeStruct((L, D), f32),
        ),
        compiler_params=_TC_PARAMS,
    )(x, bpacked, W_ae, b_ae.reshape(1, D), g_ae.reshape(1, D),
      bb_ae.reshape(1, D), Wl[0], bl[0].reshape(1, D), Wbe, b_be, g_be,
      bb_be, deg_parts)
    dinv = dinv2.reshape(N)
    norm = _norm_kernel(row, col, dinv)

    ee0, ee1 = pl.pallas_call(
        _ee_body,
        grid=(E // EB,),
        in_specs=[
            pl.BlockSpec((EB, DE), lambda i: (i, 0)),
            pl.BlockSpec((L, DE, D), lambda i: (0, 0, 0)),
            pl.BlockSpec((L, D), lambda i: (0, 0)),
        ],
        out_specs=(
            pl.BlockSpec((EB, D), lambda i: (i, 0)),
            pl.BlockSpec((EB, D), lambda i: (i, 0)),
        ),
        out_shape=(
            jax.ShapeDtypeStruct((E, D), f32),
            jax.ShapeDtypeStruct((E, D), f32),
        ),
    )(edge_attr, Wf, cf)

    ees = (ee0, ee1)
    xl = xl0
    for l in range(L):
        aggr_parts = _edge_kernel(xl, ees[l], row, col, norm, zerosD)
        last = l == L - 1
        ops = [aggr_parts, xl, Wih[l], Whh[l], bih[l].reshape(1, 3 * D),
               bhh[l].reshape(1, 3 * D), root[l].reshape(1, D), deg,
               g_bn[l].reshape(1, D), bb_bn[l].reshape(1, D)]
        if not last:
            ops += [Wl[1], bl[1].reshape(1, D)]
        xl = pl.pallas_call(
            _make_upd_body(last),
            out_shape=jax.ShapeDtypeStruct((N, D), f32),
            compiler_params=_TC_PARAMS,
        )(*ops)
    return xl


# speedup vs baseline: 7.4980x; 1.6028x over previous
import functools
import jax
import jax.numpy as jnp
from jax import lax
from jax.experimental import pallas as pl
from jax.experimental.pallas import tpu as pltpu
from jax.experimental.pallas import tpu_sc as plsc

N = 10000
E = 320000
D = 128
DE = 16
L = 2

NC = 2
NS = 16
NW = NC * NS
EPT = E // NW
C = 80
NCHUNK = EPT // C
R8 = 624
TAIL = N - R8 * NS

_MESH = plsc.VectorSubcoreMesh(core_axis_name="c", subcore_axis_name="s")
_SC_PARAMS = pltpu.CompilerParams(needs_layout_passes=False)
_TC_PARAMS = pltpu.CompilerParams(vmem_limit_bytes=100 * 1024 * 1024)


def _stripe_copy(src_at, dst_at, s):
    off = pl.multiple_of(s * R8, 8)
    pltpu.sync_copy(src_at(off, R8), dst_at(off, R8))

    @pl.when(s == NS - 1)
    def _():
        pltpu.sync_copy(src_at(R8 * NS, TAIL), dst_at(R8 * NS, TAIL))


@functools.partial(
    pl.kernel,
    mesh=_MESH,
    out_type=jax.ShapeDtypeStruct((NC, N, D), jnp.float32),
    scratch_types=[
        pltpu.VMEM((C,), jnp.int32),
        pltpu.VMEM((C, D), jnp.float32),
        pltpu.VMEM_SHARED((N, D), jnp.float32),
    ],
    compiler_params=_SC_PARAMS,
)
def _deg_kernel(row_hbm, zeros_hbm, out_hbm, rowb, onesb, deg_sh):
    c = lax.axis_index("c")
    s = lax.axis_index("s")
    wid = c * NS + s
    _stripe_copy(lambda o, n: zeros_hbm.at[pl.ds(o, n)],
                 lambda o, n: deg_sh.at[pl.ds(o, n)], s)
    one = jnp.full((16,), 1.0, jnp.float32)

    def fill(i, carry):
        for d in range(D // 16):
            onesb[i, pl.ds(d * 16, 16)] = one
        return carry

    lax.fori_loop(0, C, fill, 0)
    plsc.subcore_barrier()

    def chunk(j, carry):
        base = pl.multiple_of(wid * EPT + j * C, 8)
        pltpu.sync_copy(row_hbm.at[pl.ds(base, C)], rowb)
        pltpu.sync_copy(onesb, deg_sh.at[rowb], add=True)
        return carry

    lax.fori_loop(0, NCHUNK, chunk, 0)
    plsc.subcore_barrier()
    _stripe_copy(lambda o, n: deg_sh.at[pl.ds(o, n)],
                 lambda o, n: out_hbm.at[c, pl.ds(o, n)], s)


NB = 2000


@functools.partial(
    pl.kernel,
    mesh=_MESH,
    out_type=jax.ShapeDtypeStruct((E,), jnp.float32),
    scratch_types=[
        pltpu.VMEM((NB,), jnp.int32),
        pltpu.VMEM((NB,), jnp.int32),
        pltpu.VMEM((NB,), jnp.float32),
        pltpu.VMEM((N,), jnp.float32),
    ],
    compiler_params=_SC_PARAMS,
)
def _norm_kernel(row_hbm, col_hbm, dinv_hbm, out_hbm, rowb, colb, normb, dinvv):
    c = lax.axis_index("c")
    s = lax.axis_index("s")
    wid = c * NS + s
    pltpu.sync_copy(dinv_hbm, dinvv)

    def blk(j, carry):
        base = pl.multiple_of(wid * EPT + j * NB, 8)
        pltpu.sync_copy(row_hbm.at[pl.ds(base, NB)], rowb)
        pltpu.sync_copy(col_hbm.at[pl.ds(base, NB)], colb)

        def nloop(k, cy):
            dr = plsc.load_gather(dinvv, [rowb[pl.ds(k * 16, 16)]])
            dc = plsc.load_gather(dinvv, [colb[pl.ds(k * 16, 16)]])
            normb[pl.ds(k * 16, 16)] = dr * dc
            return cy

        lax.fori_loop(0, NB // 16, nloop, 0)
        pltpu.sync_copy(normb, out_hbm.at[pl.ds(base, NB)])
        return carry

    lax.fori_loop(0, EPT // NB, blk, 0)


@functools.partial(
    pl.kernel,
    mesh=_MESH,
    out_type=jax.ShapeDtypeStruct((NC, N, D), jnp.float32),
    scratch_types=[
        [pltpu.VMEM((C,), jnp.int32)] * 2,
        [pltpu.VMEM((C,), jnp.int32)] * 2,
        [pltpu.VMEM((C,), jnp.float32)] * 2,
        [pltpu.VMEM((C, D), jnp.float32)] * 2,
        [pltpu.VMEM((C, D), jnp.float32)] * 2,
        [pltpu.SemaphoreType.DMA] * 2,
        [pltpu.SemaphoreType.DMA] * 2,
        [pltpu.SemaphoreType.DMA] * 2,
        [pltpu.SemaphoreType.DMA] * 2,
        pltpu.VMEM_SHARED((N, D), jnp.float32),
    ],
    compiler_params=_SC_PARAMS,
)
def _edge_kernel(xl_hbm, ee_hbm, row_hbm, col_hbm, norm_hbm, zeros_hbm,
                 out_hbm, rowb, colb, normb, xg, el, gsem, esem, nsem,
                 ssem, aggr_sh):
    c = lax.axis_index("c")
    s = lax.axis_index("s")
    wid = c * NS + s
    _stripe_copy(lambda o, n: zeros_hbm.at[pl.ds(o, n)],
                 lambda o, n: aggr_sh.at[pl.ds(o, n)], s)
    plsc.subcore_barrier()

    def drain(b):
        pltpu.make_async_copy(xg[b], aggr_sh.at[colb[b]], ssem[b]).wait()

    def stage(j, b):
        base = pl.multiple_of(wid * EPT + j * C, 8)
        pltpu.sync_copy(row_hbm.at[pl.ds(base, C)], rowb[b])
        pltpu.sync_copy(col_hbm.at[pl.ds(base, C)], colb[b])
        pltpu.async_copy(xl_hbm.at[rowb[b]], xg[b], gsem[b])
        pltpu.async_copy(ee_hbm.at[pl.ds(base, C)], el[b], esem[b])
        pltpu.async_copy(norm_hbm.at[pl.ds(base, C)], normb[b], nsem[b])

    def front(j, b):
        drain(b)
        stage(j, b)

    def back(b):
        pltpu.make_async_copy(xl_hbm.at[rowb[b]], xg[b], gsem[b]).wait()
        pltpu.make_async_copy(ee_hbm.at[pl.ds(0, C)], el[b], esem[b]).wait()
        pltpu.make_async_copy(norm_hbm.at[pl.ds(0, C)], normb[b], nsem[b]).wait()

        def eloop(g, cy):
            for k in range(16):
                e = g * 16 + k
                nv = plsc.load_gather(normb[b], [jnp.full((16,), e, jnp.int32)])
                for d in range(D // 16):
                    v = xg[b][e, pl.ds(d * 16, 16)] + el[b][e, pl.ds(d * 16, 16)]
                    xg[b][e, pl.ds(d * 16, 16)] = jnp.maximum(v, 0.0) * nv
            return cy

        lax.fori_loop(0, C // 16, eloop, 0)
        pltpu.async_copy(xg[b], aggr_sh.at[colb[b]], ssem[b], add=True)

    stage(0, 0)
    stage(1, 1)
    back(0)
    front(2, 0)
    back(1)

    def pair(i, carry):
        front(2 * i + 3, 1)
        back(0)
        front(2 * i + 4, 0)
        back(1)
        return carry

    lax.fori_loop(0, (NCHUNK - 3) // 2, pair, 0)
    back(0)
    drain(1)
    drain(0)
    plsc.subcore_barrier()
    _stripe_copy(lambda o, n: aggr_sh.at[pl.ds(o, n)],
                 lambda o, n: out_hbm.at[c, pl.ds(o, n)], s)


def _prep_body(x_ref, b_ref, wae_ref, bae_ref, gae_ref, bbae_ref,
               wl0_ref, bl0_ref, wbe_ref, bbe_ref, gbe_ref, bbbe_ref,
               parts_ref, xl0_ref, deg_ref, dinv_ref, wf_ref, cf_ref):
    f32 = jnp.float32
    x = x_ref[...]
    y = jnp.dot(x, wae_ref[...], preferred_element_type=f32) + bae_ref[...]
    m = jnp.mean(y, axis=0, keepdims=True)
    v = jnp.mean((y - m) ** 2, axis=0, keepdims=True)
    h0 = (y - m) / jnp.sqrt(v + 1e-6) * gae_ref[...] + bbae_ref[...]
    xl0_ref[...] = jnp.dot(h0, wl0_ref[...], preferred_element_type=f32) + bl0_ref[...]

    parts = parts_ref[...]
    deg = parts[0, :, 0:1] + parts[1, :, 0:1] + 1.0
    deg_ref[...] = deg
    dinv_ref[...] = lax.rsqrt(deg)

    bmat = b_ref[...]
    bb = lax.dot_general(bmat, bmat, (((0,), (0,)), ((), ())),
                         preferred_element_type=f32)
    ones_row = jnp.full((1, bmat.shape[0]), 1.0, f32)
    csum = jnp.dot(ones_row, bmat, preferred_element_type=f32)
    m2 = jnp.zeros((16, 16), f32)
    asum = jnp.zeros((1, 16), f32)
    for i in range(8):
        m2 = m2 + bb[i * 16:(i + 1) * 16, i * 16:(i + 1) * 16]
        asum = asum + csum[:, i * 16:(i + 1) * 16]
    abar = asum * (1.0 / E)
    cov = m2 * (1.0 / E) - lax.dot_general(
        abar, abar, (((0,), (0,)), ((), ())), preferred_element_type=f32)

    for l in range(L):
        w = wbe_ref[l]
        mu = jnp.dot(abar, w, preferred_element_type=f32) + bbe_ref[l:l + 1, :]
        cw = jnp.dot(cov, w, preferred_element_type=f32)
        var = jnp.sum(w * cw, axis=0, keepdims=True)
        sc = gbe_ref[l:l + 1, :] / jnp.sqrt(var + 1e-6)
        wf_ref[l] = w * sc
        cf_ref[l:l + 1, :] = (bbe_ref[l:l + 1, :] - mu) * sc + bbbe_ref[l:l + 1, :]


EB = 8000


def _ee_body(a_ref, wf_ref, cf_ref, ee0_ref, ee1_ref):
    f32 = jnp.float32
    a = a_ref[...]
    ee0_ref[...] = jnp.dot(a, wf_ref[0], preferred_element_type=f32) + cf_ref[0:1, :]
    ee1_ref[...] = jnp.dot(a, wf_ref[1], preferred_element_type=f32) + cf_ref[1:2, :]


def _make_upd_body(last):
    def body(parts_ref, xl_ref, wih_ref, whh_ref, bih_ref, bhh_ref,
             root_ref, deg_ref, gbn_ref, bbbn_ref, *rest):
        f32 = jnp.float32
        if last:
            (out_ref,) = rest
        else:
            wl1_ref, bl1_ref, out_ref = rest
        parts = parts_ref[...]
        aggr = parts[0] + parts[1]
        xl = xl_ref[...]
        gi = jnp.dot(aggr, wih_ref[...], preferred_element_type=f32) + bih_ref[...]
        gh = jnp.dot(xl, whh_ref[...], preferred_element_type=f32) + bhh_ref[...]
        r = jax.nn.sigmoid(gi[:, 0:D] + gh[:, 0:D])
        z = jax.nn.sigmoid(gi[:, D:2 * D] + gh[:, D:2 * D])
        n = jnp.tanh(gi[:, 2 * D:3 * D] + r * gh[:, 2 * D:3 * D])
        upd = (1.0 - z) * n + z * xl
        conv = upd + jnp.maximum(xl + root_ref[...], 0.0) / deg_ref[...]
        m = jnp.mean(conv, axis=0, keepdims=True)
        v = jnp.mean((conv - m) ** 2, axis=0, keepdims=True)
        hb = (conv - m) / jnp.sqrt(v + 1e-5) * gbn_ref[...] + bbbn_ref[...]
        if last:
            out_ref[...] = hb
        else:
            h = jnp.maximum(hb, 0.0)
            out_ref[...] = jnp.dot(h, wl1_ref[...], preferred_element_type=f32) + bl1_ref[...]
    return body


def kernel(x, edge_index, edge_attr, W_ae, b_ae, g_ae, bb_ae, Wl, bl, root,
           Wbe, b_be, g_be, bb_be, Wih, Whh, bih, bhh, g_bn, bb_bn):
    f32 = jnp.float32
    zerosD = jnp.zeros((N, D), f32)
    bpacked = edge_attr.reshape(E * DE // 128, 128)
    row = edge_index[0]
    col = edge_index[1]

    deg_parts = _deg_kernel(row, zerosD)

    xl0, deg, dinv2, Wf, cf = pl.pallas_call(
        _prep_body,
        out_shape=(
            jax.ShapeDtypeStruct((N, D), f32),
            jax.ShapeDtypeStruct((N, 1), f32),
            jax.ShapeDtypeStruct((N, 1), f32),
            jax.ShapeDtypeStruct((L, DE, D), f32),
            jax.ShapeDtypeStruct((L, D), f32),
        ),
        compiler_params=_TC_PARAMS,
    )(x, bpacked, W_ae, b_ae.reshape(1, D), g_ae.reshape(1, D),
      bb_ae.reshape(1, D), Wl[0], bl[0].reshape(1, D), Wbe, b_be, g_be,
      bb_be, deg_parts)
    dinv = dinv2.reshape(N)
    norm = _norm_kernel(row, col, dinv)

    ee0, ee1 = pl.pallas_call(
        _ee_body,
        grid=(E // EB,),
        in_specs=[
            pl.BlockSpec((EB, DE), lambda i: (i, 0)),
            pl.BlockSpec((L, DE, D), lambda i: (0, 0, 0)),
            pl.BlockSpec((L, D), lambda i: (0, 0)),
        ],
        out_specs=(
            pl.BlockSpec((EB, D), lambda i: (i, 0)),
            pl.BlockSpec((EB, D), lambda i: (i, 0)),
        ),
        out_shape=(
            jax.ShapeDtypeStruct((E, D), f32),
            jax.ShapeDtypeStruct((E, D), f32),
        ),
    )(edge_attr, Wf, cf)

    ees = (ee0, ee1)
    xl = xl0
    for l in range(L):
        aggr_parts = _edge_kernel(xl, ees[l], row, col, norm, zerosD)
        last = l == L - 1
        ops = [aggr_parts, xl, Wih[l], Whh[l], bih[l].reshape(1, 3 * D),
               bhh[l].reshape(1, 3 * D), root[l].reshape(1, D), deg,
               g_bn[l].reshape(1, D), bb_bn[l].reshape(1, D)]
        if not last:
            ops += [Wl[1], bl[1].reshape(1, D)]
        xl = pl.pallas_call(
            _make_upd_body(last),
            out_shape=jax.ShapeDtypeStruct((N, D), f32),
            compiler_params=_TC_PARAMS,
        )(*ops)
    return xl


# speedup vs baseline: 8.2146x; 1.0956x over previous
import functools
import jax
import jax.numpy as jnp
from jax import lax
from jax.experimental import pallas as pl
from jax.experimental.pallas import tpu as pltpu
from jax.experimental.pallas import tpu_sc as plsc

N = 10000
E = 320000
D = 128
DE = 16
L = 2

NC = 2
NS = 16
NW = NC * NS
EPT = E // NW
C = 80
NCHUNK = EPT // C
R8 = 624
TAIL = N - R8 * NS

_MESH = plsc.VectorSubcoreMesh(core_axis_name="c", subcore_axis_name="s")
_SC_PARAMS = pltpu.CompilerParams(needs_layout_passes=False)
_TC_PARAMS = pltpu.CompilerParams(vmem_limit_bytes=100 * 1024 * 1024)


def _stripe_copy(src_at, dst_at, s):
    off = pl.multiple_of(s * R8, 8)
    pltpu.sync_copy(src_at(off, R8), dst_at(off, R8))

    @pl.when(s == NS - 1)
    def _():
        pltpu.sync_copy(src_at(R8 * NS, TAIL), dst_at(R8 * NS, TAIL))


@functools.partial(
    pl.kernel,
    mesh=_MESH,
    out_type=jax.ShapeDtypeStruct((NC, N, D), jnp.float32),
    scratch_types=[
        pltpu.VMEM((C,), jnp.int32),
        pltpu.VMEM((C, D), jnp.float32),
        pltpu.VMEM_SHARED((N, D), jnp.float32),
    ],
    compiler_params=_SC_PARAMS,
)
def _deg_kernel(row_hbm, zeros_hbm, out_hbm, rowb, onesb, deg_sh):
    c = lax.axis_index("c")
    s = lax.axis_index("s")
    wid = c * NS + s
    _stripe_copy(lambda o, n: zeros_hbm.at[pl.ds(o, n)],
                 lambda o, n: deg_sh.at[pl.ds(o, n)], s)
    one = jnp.full((16,), 1.0, jnp.float32)

    def fill(i, carry):
        for d in range(D // 16):
            onesb[i, pl.ds(d * 16, 16)] = one
        return carry

    lax.fori_loop(0, C, fill, 0)
    plsc.subcore_barrier()

    def chunk(j, carry):
        base = pl.multiple_of(wid * EPT + j * C, 8)
        pltpu.sync_copy(row_hbm.at[pl.ds(base, C)], rowb)
        pltpu.sync_copy(onesb, deg_sh.at[rowb], add=True)
        return carry

    lax.fori_loop(0, NCHUNK, chunk, 0)
    plsc.subcore_barrier()
    _stripe_copy(lambda o, n: deg_sh.at[pl.ds(o, n)],
                 lambda o, n: out_hbm.at[c, pl.ds(o, n)], s)


NB = 2000


@functools.partial(
    pl.kernel,
    mesh=_MESH,
    out_type=jax.ShapeDtypeStruct((E,), jnp.float32),
    scratch_types=[
        pltpu.VMEM((NB,), jnp.int32),
        pltpu.VMEM((NB,), jnp.int32),
        pltpu.VMEM((NB,), jnp.float32),
        pltpu.VMEM((N,), jnp.float32),
    ],
    compiler_params=_SC_PARAMS,
)
def _norm_kernel(row_hbm, col_hbm, dinv_hbm, out_hbm, rowb, colb, normb, dinvv):
    c = lax.axis_index("c")
    s = lax.axis_index("s")
    wid = c * NS + s
    pltpu.sync_copy(dinv_hbm, dinvv)

    def blk(j, carry):
        base = pl.multiple_of(wid * EPT + j * NB, 8)
        pltpu.sync_copy(row_hbm.at[pl.ds(base, NB)], rowb)
        pltpu.sync_copy(col_hbm.at[pl.ds(base, NB)], colb)

        def nloop(k, cy):
            dr = plsc.load_gather(dinvv, [rowb[pl.ds(k * 16, 16)]])
            dc = plsc.load_gather(dinvv, [colb[pl.ds(k * 16, 16)]])
            normb[pl.ds(k * 16, 16)] = dr * dc
            return cy

        lax.fori_loop(0, NB // 16, nloop, 0)
        pltpu.sync_copy(normb, out_hbm.at[pl.ds(base, NB)])
        return carry

    lax.fori_loop(0, EPT // NB, blk, 0)


@functools.partial(
    pl.kernel,
    mesh=_MESH,
    out_type=jax.ShapeDtypeStruct((NC, N, D), jnp.float32),
    scratch_types=[
        [pltpu.VMEM((C,), jnp.int32)] * 2,
        [pltpu.VMEM((C,), jnp.int32)] * 2,
        [pltpu.VMEM((C,), jnp.float32)] * 2,
        [pltpu.VMEM((C, D), jnp.float32)] * 2,
        [pltpu.VMEM((C, D), jnp.float32)] * 2,
        [pltpu.SemaphoreType.DMA] * 2,
        [pltpu.SemaphoreType.DMA] * 2,
        [pltpu.SemaphoreType.DMA] * 2,
        [pltpu.SemaphoreType.DMA] * 2,
        pltpu.VMEM_SHARED((N, D), jnp.float32),
    ],
    compiler_params=_SC_PARAMS,
)
def _edge_kernel(xl_hbm, ee_hbm, row_hbm, col_hbm, norm_hbm, zeros_hbm,
                 out_hbm, rowb, colb, normb, xg, el, gsem, esem, nsem,
                 ssem, aggr_sh):
    c = lax.axis_index("c")
    s = lax.axis_index("s")
    wid = c * NS + s
    _stripe_copy(lambda o, n: zeros_hbm.at[pl.ds(o, n)],
                 lambda o, n: aggr_sh.at[pl.ds(o, n)], s)
    plsc.subcore_barrier()

    def drain(b):
        pltpu.make_async_copy(xg[b], aggr_sh.at[colb[b]], ssem[b]).wait()

    def stage(j, b):
        base = pl.multiple_of(wid * EPT + j * C, 8)
        pltpu.sync_copy(row_hbm.at[pl.ds(base, C)], rowb[b])
        pltpu.sync_copy(col_hbm.at[pl.ds(base, C)], colb[b])
        pltpu.async_copy(xl_hbm.at[rowb[b]], xg[b], gsem[b])
        pltpu.async_copy(ee_hbm.at[pl.ds(base, C)], el[b], esem[b])
        pltpu.async_copy(norm_hbm.at[pl.ds(base, C)], normb[b], nsem[b])

    def front(j, b):
        drain(b)
        stage(j, b)

    def back(b):
        pltpu.make_async_copy(xl_hbm.at[rowb[b]], xg[b], gsem[b]).wait()
        pltpu.make_async_copy(ee_hbm.at[pl.ds(0, C)], el[b], esem[b]).wait()
        pltpu.make_async_copy(norm_hbm.at[pl.ds(0, C)], normb[b], nsem[b]).wait()

        @plsc.parallel_loop(0, C, step=2, unroll=2)
        def eloop(e0):
            for k in range(2):
                e = e0 + k
                nv = plsc.load_gather(normb[b], [jnp.full((16,), e, jnp.int32)])
                for d in range(D // 16):
                    v = xg[b][e, pl.ds(d * 16, 16)] + el[b][e, pl.ds(d * 16, 16)]
                    xg[b][e, pl.ds(d * 16, 16)] = jnp.maximum(v, 0.0) * nv
        pltpu.async_copy(xg[b], aggr_sh.at[colb[b]], ssem[b], add=True)

    stage(0, 0)
    stage(1, 1)
    back(0)
    front(2, 0)
    back(1)

    def pair(i, carry):
        front(2 * i + 3, 1)
        back(0)
        front(2 * i + 4, 0)
        back(1)
        return carry

    lax.fori_loop(0, (NCHUNK - 3) // 2, pair, 0)
    back(0)
    drain(1)
    drain(0)
    plsc.subcore_barrier()
    _stripe_copy(lambda o, n: aggr_sh.at[pl.ds(o, n)],
                 lambda o, n: out_hbm.at[c, pl.ds(o, n)], s)


def _prep_body(x_ref, b_ref, wae_ref, bae_ref, gae_ref, bbae_ref,
               wl0_ref, bl0_ref, wbe_ref, bbe_ref, gbe_ref, bbbe_ref,
               parts_ref, xl0_ref, deg_ref, dinv_ref, wf_ref, cf_ref):
    f32 = jnp.float32
    x = x_ref[...]
    y = jnp.dot(x, wae_ref[...], preferred_element_type=f32) + bae_ref[...]
    m = jnp.mean(y, axis=0, keepdims=True)
    v = jnp.mean((y - m) ** 2, axis=0, keepdims=True)
    h0 = (y - m) / jnp.sqrt(v + 1e-6) * gae_ref[...] + bbae_ref[...]
    xl0_ref[...] = jnp.dot(h0, wl0_ref[...], preferred_element_type=f32) + bl0_ref[...]

    parts = parts_ref[...]
    deg = parts[0, :, 0:1] + parts[1, :, 0:1] + 1.0
    deg_ref[...] = deg
    dinv_ref[...] = lax.rsqrt(deg)

    bmat = b_ref[...]
    bb = lax.dot_general(bmat, bmat, (((0,), (0,)), ((), ())),
                         preferred_element_type=f32)
    ones_row = jnp.full((1, bmat.shape[0]), 1.0, f32)
    csum = jnp.dot(ones_row, bmat, preferred_element_type=f32)
    m2 = jnp.zeros((16, 16), f32)
    asum = jnp.zeros((1, 16), f32)
    for i in range(8):
        m2 = m2 + bb[i * 16:(i + 1) * 16, i * 16:(i + 1) * 16]
        asum = asum + csum[:, i * 16:(i + 1) * 16]
    abar = asum * (1.0 / E)
    cov = m2 * (1.0 / E) - lax.dot_general(
        abar, abar, (((0,), (0,)), ((), ())), preferred_element_type=f32)

    for l in range(L):
        w = wbe_ref[l]
        mu = jnp.dot(abar, w, preferred_element_type=f32) + bbe_ref[l:l + 1, :]
        cw = jnp.dot(cov, w, preferred_element_type=f32)
        var = jnp.sum(w * cw, axis=0, keepdims=True)
        sc = gbe_ref[l:l + 1, :] / jnp.sqrt(var + 1e-6)
        wf_ref[l] = w * sc
        cf_ref[l:l + 1, :] = (bbe_ref[l:l + 1, :] - mu) * sc + bbbe_ref[l:l + 1, :]


EB = 8000


def _ee_body(a_ref, wf_ref, cf_ref, ee0_ref, ee1_ref):
    f32 = jnp.float32
    a = a_ref[...]
    ee0_ref[...] = jnp.dot(a, wf_ref[0], preferred_element_type=f32) + cf_ref[0:1, :]
    ee1_ref[...] = jnp.dot(a, wf_ref[1], preferred_element_type=f32) + cf_ref[1:2, :]


def _make_upd_body(last):
    def body(parts_ref, xl_ref, wih_ref, whh_ref, bih_ref, bhh_ref,
             root_ref, deg_ref, gbn_ref, bbbn_ref, *rest):
        f32 = jnp.float32
        if last:
            (out_ref,) = rest
        else:
            wl1_ref, bl1_ref, out_ref = rest
        parts = parts_ref[...]
        aggr = parts[0] + parts[1]
        xl = xl_ref[...]
        gi = jnp.dot(aggr, wih_ref[...], preferred_element_type=f32) + bih_ref[...]
        gh = jnp.dot(xl, whh_ref[...], preferred_element_type=f32) + bhh_ref[...]
        r = jax.nn.sigmoid(gi[:, 0:D] + gh[:, 0:D])
        z = jax.nn.sigmoid(gi[:, D:2 * D] + gh[:, D:2 * D])
        n = jnp.tanh(gi[:, 2 * D:3 * D] + r * gh[:, 2 * D:3 * D])
        upd = (1.0 - z) * n + z * xl
        conv = upd + jnp.maximum(xl + root_ref[...], 0.0) / deg_ref[...]
        m = jnp.mean(conv, axis=0, keepdims=True)
        v = jnp.mean((conv - m) ** 2, axis=0, keepdims=True)
        hb = (conv - m) / jnp.sqrt(v + 1e-5) * gbn_ref[...] + bbbn_ref[...]
        if last:
            out_ref[...] = hb
        else:
            h = jnp.maximum(hb, 0.0)
            out_ref[...] = jnp.dot(h, wl1_ref[...], preferred_element_type=f32) + bl1_ref[...]
    return body


def kernel(x, edge_index, edge_attr, W_ae, b_ae, g_ae, bb_ae, Wl, bl, root,
           Wbe, b_be, g_be, bb_be, Wih, Whh, bih, bhh, g_bn, bb_bn):
    f32 = jnp.float32
    zerosD = jnp.zeros((N, D), f32)
    bpacked = edge_attr.reshape(E * DE // 128, 128)
    row = edge_index[0]
    col = edge_index[1]

    deg_parts = _deg_kernel(row, zerosD)

    xl0, deg, dinv2, Wf, cf = pl.pallas_call(
        _prep_body,
        out_shape=(
            jax.ShapeDtypeStruct((N, D), f32),
            jax.ShapeDtypeStruct((N, 1), f32),
            jax.ShapeDtypeStruct((N, 1), f32),
            jax.ShapeDtypeStruct((L, DE, D), f32),
            jax.ShapeDtypeStruct((L, D), f32),
        ),
        compiler_params=_TC_PARAMS,
    )(x, bpacked, W_ae, b_ae.reshape(1, D), g_ae.reshape(1, D),
      bb_ae.reshape(1, D), Wl[0], bl[0].reshape(1, D), Wbe, b_be, g_be,
      bb_be, deg_parts)
    dinv = dinv2.reshape(N)
    norm = _norm_kernel(row, col, dinv)

    ee0, ee1 = pl.pallas_call(
        _ee_body,
        grid=(E // EB,),
        in_specs=[
            pl.BlockSpec((EB, DE), lambda i: (i, 0)),
            pl.BlockSpec((L, DE, D), lambda i: (0, 0, 0)),
            pl.BlockSpec((L, D), lambda i: (0, 0)),
        ],
        out_specs=(
            pl.BlockSpec((EB, D), lambda i: (i, 0)),
            pl.BlockSpec((EB, D), lambda i: (i, 0)),
        ),
        out_shape=(
            jax.ShapeDtypeStruct((E, D), f32),
            jax.ShapeDtypeStruct((E, D), f32),
        ),
    )(edge_attr, Wf, cf)

    ees = (ee0, ee1)
    xl = xl0
    for l in range(L):
        aggr_parts = _edge_kernel(xl, ees[l], row, col, norm, zerosD)
        last = l == L - 1
        ops = [aggr_parts, xl, Wih[l], Whh[l], bih[l].reshape(1, 3 * D),
               bhh[l].reshape(1, 3 * D), root[l].reshape(1, D), deg,
               g_bn[l].reshape(1, D), bb_bn[l].reshape(1, D)]
        if not last:
            ops += [Wl[1], bl[1].reshape(1, D)]
        xl = pl.pallas_call(
            _make_upd_body(last),
            out_shape=jax.ShapeDtypeStruct((N, D), f32),
            compiler_params=_TC_PARAMS,
        )(*ops)
    return xl
